# Initial kernel scaffold; baseline (speedup 1.0000x reference)
import jax
import jax.numpy as jnp
from jax.experimental import pallas as pl


def kernel(x, edge_index, W1, att_src1, att_dst1, b1, W2, att_src2, att_dst2, b2, Wc, bc):
    raise NotImplementedError("write your pallas kernel here")


# speedup vs baseline: 43.7907x; 43.7907x over previous
import functools

import jax
import jax.numpy as jnp
from jax import lax
from jax.experimental import pallas as pl
from jax.experimental.pallas import tpu as pltpu
from jax.experimental.pallas import tpu_sc as plsc

N = 10000
E = 320000
F = 128
HEADS = 4
HID = 32

N_PAD = 10240
CHUNK = 128
N_WORKERS = 32
CPW = 79
E_PAD = N_WORKERS * CPW * CHUNK
ROWS_PER_SUB = N_PAD // 16


def _leaky(x):
    return jnp.where(x >= 0.0, x, 0.2 * x)


def _elu(x):
    return jnp.where(x > 0.0, x, jnp.exp(x) - 1.0)


def _prologue_body(x_ref, w_ref, ms_ref, md_ref, h_ref, as_ref, ad_ref):
    h = jnp.dot(x_ref[...], w_ref[...], preferred_element_type=jnp.float32)
    h_ref[...] = h
    as_ref[...] = jnp.dot(h, ms_ref[...], preferred_element_type=jnp.float32)
    ad_ref[...] = jnp.dot(h, md_ref[...], preferred_element_type=jnp.float32)


def _prologue(x_pad, W, Msrc, Mdst):
    blk = 1024
    grid = N_PAD // blk
    return pl.pallas_call(
        _prologue_body,
        grid=(grid,),
        in_specs=[
            pl.BlockSpec((blk, F), lambda i: (i, 0)),
            pl.BlockSpec((F, F), lambda i: (0, 0)),
            pl.BlockSpec((F, 16), lambda i: (0, 0)),
            pl.BlockSpec((F, 16), lambda i: (0, 0)),
        ],
        out_specs=[
            pl.BlockSpec((blk, F), lambda i: (i, 0)),
            pl.BlockSpec((blk, 16), lambda i: (i, 0)),
            pl.BlockSpec((blk, 16), lambda i: (i, 0)),
        ],
        out_shape=[
            jax.ShapeDtypeStruct((N_PAD, F), jnp.float32),
            jax.ShapeDtypeStruct((N_PAD, 16), jnp.float32),
            jax.ShapeDtypeStruct((N_PAD, 16), jnp.float32),
        ],
    )(x_pad, W, Msrc, Mdst)


def _edge_pass(H, As, Ad, srcp, dstp):
    mesh = plsc.VectorSubcoreMesh(core_axis_name="c", subcore_axis_name="s")

    @functools.partial(
        pl.kernel,
        out_type=[
            jax.ShapeDtypeStruct((2, N_PAD, F), jnp.float32),
            jax.ShapeDtypeStruct((2, N_PAD, 16), jnp.float32),
        ],
        mesh=mesh,
        compiler_params=pltpu.CompilerParams(use_tc_tiling_on_sc=False),
        scratch_types=[
            pltpu.VMEM((CHUNK,), jnp.int32),
            pltpu.VMEM((CHUNK,), jnp.int32),
            pltpu.VMEM((CHUNK, F), jnp.float32),
            pltpu.VMEM((CHUNK, 16), jnp.float32),
            pltpu.VMEM((CHUNK, 16), jnp.float32),
            pltpu.VMEM_SHARED((N_PAD, F), jnp.float32),
            pltpu.VMEM_SHARED((N_PAD, 16), jnp.float32),
            pltpu.SemaphoreType.DMA,
            pltpu.SemaphoreType.DMA,
            pltpu.SemaphoreType.DMA,
        ],
    )
    def k(h_r, as_r, ad_r, src_r, dst_r, accm_o, accw_o,
          idx_s, idx_d, rows, asr, adr, accm_sh, accw_sh,
          sem_r, sem_a, sem_d):
        cid = lax.axis_index("c")
        sid = lax.axis_index("s")
        wid = sid * 2 + cid

        z16 = jnp.zeros((16,), jnp.float32)

        def _zrow(i, carry):
            for j in range(F // 16):
                rows[i, pl.ds(j * 16, 16)] = z16
            asr[i, :] = z16
            return carry
        lax.fori_loop(0, CHUNK, _zrow, 0)
        for t in range(ROWS_PER_SUB // CHUNK):
            r0 = sid * ROWS_PER_SUB + t * CHUNK
            pltpu.sync_copy(rows, accm_sh.at[pl.ds(r0, CHUNK)])
            pltpu.sync_copy(asr, accw_sh.at[pl.ds(r0, CHUNK)])
        plsc.subcore_barrier()

        def _chunk(j, carry):
            base = (wid * CPW + j) * CHUNK
            pltpu.sync_copy(src_r.at[pl.ds(base, CHUNK)], idx_s)
            pltpu.sync_copy(dst_r.at[pl.ds(base, CHUNK)], idx_d)
            cp_r = pltpu.async_copy(h_r.at[idx_s], rows, sem_r)
            cp_a = pltpu.async_copy(as_r.at[idx_s], asr, sem_a)
            cp_d = pltpu.async_copy(ad_r.at[idx_d], adr, sem_d)
            cp_r.wait()
            cp_a.wait()
            cp_d.wait()

            def _edge(e, carry2):
                wrow = jnp.exp(_leaky(asr[e, :] + adr[e, :]))
                asr[e, :] = wrow
                for jb in range(F // 16):
                    s = wrow[jb * 16 // HID]
                    rows[e, pl.ds(jb * 16, 16)] = rows[e, pl.ds(jb * 16, 16)] * s
                return carry2
            lax.fori_loop(0, CHUNK, _edge, 0)

            pltpu.sync_copy(rows, accm_sh.at[idx_d], add=True)
            pltpu.sync_copy(asr, accw_sh.at[idx_d], add=True)
            return carry
        lax.fori_loop(0, CPW, _chunk, 0)

        plsc.subcore_barrier()
        r0 = sid * ROWS_PER_SUB
        pltpu.sync_copy(accm_sh.at[pl.ds(r0, ROWS_PER_SUB)],
                        accm_o.at[cid, pl.ds(r0, ROWS_PER_SUB)])
        pltpu.sync_copy(accw_sh.at[pl.ds(r0, ROWS_PER_SUB)],
                        accw_o.at[cid, pl.ds(r0, ROWS_PER_SUB)])

    return k(H, As, Ad, srcp, dstp)


def _mid_body(am_ref, aw_ref, h_ref, as_ref, ad_ref, erep_ref, b_ref,
              w2_ref, ms_ref, md_ref, h2_ref, as2_ref, ad2_ref):
    wself = jnp.exp(_leaky(as_ref[...] + ad_ref[...]))
    erep = erep_ref[...]
    den = jnp.dot(aw_ref[0] + aw_ref[1] + wself, erep,
                  preferred_element_type=jnp.float32)
    num = (am_ref[0] + am_ref[1]
           + jnp.dot(wself, erep, preferred_element_type=jnp.float32)
           * h_ref[...])
    x2 = _elu(num / den + b_ref[...])
    h2 = jnp.dot(x2, w2_ref[...], preferred_element_type=jnp.float32)
    h2_ref[...] = h2
    as2_ref[...] = jnp.dot(h2, ms_ref[...], preferred_element_type=jnp.float32)
    ad2_ref[...] = jnp.dot(h2, md_ref[...], preferred_element_type=jnp.float32)


def _mid(accm, accw, H1, As1, Ad1, Erep, b1, W2, Msrc2, Mdst2):
    blk = 1024
    grid = N_PAD // blk
    return pl.pallas_call(
        _mid_body,
        grid=(grid,),
        in_specs=[
            pl.BlockSpec((2, blk, F), lambda i: (0, i, 0)),
            pl.BlockSpec((2, blk, 16), lambda i: (0, i, 0)),
            pl.BlockSpec((blk, F), lambda i: (i, 0)),
            pl.BlockSpec((blk, 16), lambda i: (i, 0)),
            pl.BlockSpec((blk, 16), lambda i: (i, 0)),
            pl.BlockSpec((16, F), lambda i: (0, 0)),
            pl.BlockSpec((1, F), lambda i: (0, 0)),
            pl.BlockSpec((F, F), lambda i: (0, 0)),
            pl.BlockSpec((F, 16), lambda i: (0, 0)),
            pl.BlockSpec((F, 16), lambda i: (0, 0)),
        ],
        out_specs=[
            pl.BlockSpec((blk, F), lambda i: (i, 0)),
            pl.BlockSpec((blk, 16), lambda i: (i, 0)),
            pl.BlockSpec((blk, 16), lambda i: (i, 0)),
        ],
        out_shape=[
            jax.ShapeDtypeStruct((N_PAD, F), jnp.float32),
            jax.ShapeDtypeStruct((N_PAD, 16), jnp.float32),
            jax.ShapeDtypeStruct((N_PAD, 16), jnp.float32),
        ],
    )(accm, accw, H1, As1, Ad1, Erep, b1, W2, Msrc2, Mdst2)


def _final_body(am_ref, aw_ref, h_ref, as_ref, ad_ref, erep_ref, mavg_ref,
                b_ref, wc_ref, bc_ref, out_ref):
    wself = jnp.exp(_leaky(as_ref[...] + ad_ref[...]))
    erep = erep_ref[...]
    den = jnp.dot(aw_ref[0] + aw_ref[1] + wself, erep,
                  preferred_element_type=jnp.float32)
    num = (am_ref[0] + am_ref[1]
           + jnp.dot(wself, erep, preferred_element_type=jnp.float32)
           * h_ref[...])
    v = num / den
    y = _elu(jnp.dot(v, mavg_ref[...], preferred_element_type=jnp.float32)
             + b_ref[...])
    out_ref[...] = (jnp.dot(y, wc_ref[...], preferred_element_type=jnp.float32)
                    + bc_ref[...])


def _final(accm, accw, H2, As2, Ad2, Erep, Mavg, b2, Wc, bc):
    blk = 1000
    grid = N // blk
    return pl.pallas_call(
        _final_body,
        grid=(grid,),
        in_specs=[
            pl.BlockSpec((2, blk, F), lambda i: (0, i, 0)),
            pl.BlockSpec((2, blk, 16), lambda i: (0, i, 0)),
            pl.BlockSpec((blk, F), lambda i: (i, 0)),
            pl.BlockSpec((blk, 16), lambda i: (i, 0)),
            pl.BlockSpec((blk, 16), lambda i: (i, 0)),
            pl.BlockSpec((16, F), lambda i: (0, 0)),
            pl.BlockSpec((F, HID), lambda i: (0, 0)),
            pl.BlockSpec((1, HID), lambda i: (0, 0)),
            pl.BlockSpec((HID, 16), lambda i: (0, 0)),
            pl.BlockSpec((1, 16), lambda i: (0, 0)),
        ],
        out_specs=pl.BlockSpec((blk, 16), lambda i: (i, 0)),
        out_shape=jax.ShapeDtypeStruct((N, 16), jnp.float32),
    )(accm, accw, H2, As2, Ad2, Erep, Mavg, b2, Wc, bc)


def kernel(x, edge_index, W1, att_src1, att_dst1, b1,
           W2, att_src2, att_dst2, b2, Wc, bc):
    f32 = jnp.float32
    hm = (jnp.arange(16)[None, :] == (jnp.arange(F) // HID)[:, None]).astype(f32)
    Erep = hm.T
    Mavg = jnp.tile(jnp.eye(HID, dtype=f32), (HEADS, 1)) * (1.0 / HEADS)
    Msrc1 = att_src1.reshape(F)[:, None] * hm
    Mdst1 = att_dst1.reshape(F)[:, None] * hm
    Msrc2 = att_src2.reshape(F)[:, None] * hm
    Mdst2 = att_dst2.reshape(F)[:, None] * hm

    x_pad = jnp.pad(x, ((0, N_PAD - N), (0, 0)))
    pad_idx = jnp.full((E_PAD - E,), N, jnp.int32)
    srcp = jnp.concatenate([edge_index[0], pad_idx])
    dstp = jnp.concatenate([edge_index[1], pad_idx])

    H1, As1, Ad1 = _prologue(x_pad, W1, Msrc1, Mdst1)
    accm1, accw1 = _edge_pass(H1, As1, Ad1, srcp, dstp)
    H2, As2, Ad2 = _mid(accm1, accw1, H1, As1, Ad1, Erep,
                        b1.reshape(1, F), W2, Msrc2, Mdst2)
    accm2, accw2 = _edge_pass(H2, As2, Ad2, srcp, dstp)
    return _final(accm2, accw2, H2, As2, Ad2, Erep, Mavg,
                  b2.reshape(1, HID), Wc, bc.reshape(1, 16))


# speedup vs baseline: 49.0607x; 1.1203x over previous
import functools

import jax
import jax.numpy as jnp
from jax import lax
from jax.experimental import pallas as pl
from jax.experimental.pallas import tpu as pltpu
from jax.experimental.pallas import tpu_sc as plsc

N = 10000
E = 320000
F = 128
HEADS = 4
HID = 32

N_PAD = 10240
CHUNK = 112
N_WORKERS = 32
CPW = 92
E_PAD = N_WORKERS * CPW * CHUNK
ROWS_PER_SUB = N_PAD // 16


def _leaky(x):
    return jnp.where(x >= 0.0, x, 0.2 * x)


def _elu(x):
    return jnp.where(x > 0.0, x, jnp.exp(x) - 1.0)


def _prologue_body(x_ref, w_ref, ms_ref, md_ref, h_ref, as_ref, ad_ref):
    h = jnp.dot(x_ref[...], w_ref[...], preferred_element_type=jnp.float32)
    h_ref[...] = h
    as_ref[...] = jnp.dot(h, ms_ref[...], preferred_element_type=jnp.float32)
    ad_ref[...] = jnp.dot(h, md_ref[...], preferred_element_type=jnp.float32)


def _prologue(x_pad, W, Msrc, Mdst):
    blk = 1024
    grid = N_PAD // blk
    return pl.pallas_call(
        _prologue_body,
        grid=(grid,),
        in_specs=[
            pl.BlockSpec((blk, F), lambda i: (i, 0)),
            pl.BlockSpec((F, F), lambda i: (0, 0)),
            pl.BlockSpec((F, 16), lambda i: (0, 0)),
            pl.BlockSpec((F, 16), lambda i: (0, 0)),
        ],
        out_specs=[
            pl.BlockSpec((blk, F), lambda i: (i, 0)),
            pl.BlockSpec((blk, 16), lambda i: (i, 0)),
            pl.BlockSpec((blk, 16), lambda i: (i, 0)),
        ],
        out_shape=[
            jax.ShapeDtypeStruct((N_PAD, F), jnp.float32),
            jax.ShapeDtypeStruct((N_PAD, 16), jnp.float32),
            jax.ShapeDtypeStruct((N_PAD, 16), jnp.float32),
        ],
    )(x_pad, W, Msrc, Mdst)


def _edge_pass(H, As, Ad, srcp, dstp):
    mesh = plsc.VectorSubcoreMesh(core_axis_name="c", subcore_axis_name="s")

    @functools.partial(
        pl.kernel,
        out_type=[
            jax.ShapeDtypeStruct((2, N_PAD, F), jnp.float32),
            jax.ShapeDtypeStruct((2, N_PAD, 16), jnp.float32),
        ],
        mesh=mesh,
        compiler_params=pltpu.CompilerParams(use_tc_tiling_on_sc=False),
        scratch_types=(
            [pltpu.VMEM((CHUNK,), jnp.int32)] * 8
            + [pltpu.VMEM((CHUNK, F), jnp.float32)] * 2
            + [pltpu.VMEM((CHUNK, 16), jnp.float32)] * 4
            + [pltpu.VMEM_SHARED((N_PAD, F), jnp.float32),
               pltpu.VMEM_SHARED((N_PAD, 16), jnp.float32)]
            + [pltpu.SemaphoreType.DMA] * 6
        ),
    )
    def k(h_r, as_r, ad_r, src_r, dst_r, accm_o, accw_o,
          is0, is1, is2, is3, id0, id1, id2, id3,
          rows0, rows1, asr0, asr1, adr0, adr1,
          accm_sh, accw_sh,
          si0, si1, si2, si3, sg0, sg1):
        cid = lax.axis_index("c")
        sid = lax.axis_index("s")
        wid = sid * 2 + cid
        e0 = wid * CPW * CHUNK

        z16 = jnp.zeros((16,), jnp.float32)

        def _zrow(i, carry):
            for j in range(F // 16):
                rows0[i, pl.ds(j * 16, 16)] = z16
            asr0[i, :] = z16
            return carry
        lax.fori_loop(0, CHUNK, _zrow, 0)
        nfull = ROWS_PER_SUB // CHUNK
        for t in range(nfull):
            r0 = sid * ROWS_PER_SUB + t * CHUNK
            pltpu.sync_copy(rows0, accm_sh.at[pl.ds(r0, CHUNK)])
            pltpu.sync_copy(asr0, accw_sh.at[pl.ds(r0, CHUNK)])
        rem = ROWS_PER_SUB - nfull * CHUNK
        if rem:
            r0 = sid * ROWS_PER_SUB + nfull * CHUNK
            pltpu.sync_copy(rows0.at[pl.ds(0, rem)],
                            accm_sh.at[pl.ds(r0, rem)])
            pltpu.sync_copy(asr0.at[pl.ds(0, rem)],
                            accw_sh.at[pl.ds(r0, rem)])
        plsc.subcore_barrier()

        isl = (is0, is1, is2, is3)
        idl = (id0, id1, id2, id3)
        sis = (si0, si1, si2, si3)
        dbufs = ((rows0, asr0, adr0, sg0), (rows1, asr1, adr1, sg1))

        def _issue_idx(j, r):
            base = e0 + j * CHUNK
            pltpu.async_copy(src_r.at[pl.ds(base, CHUNK)], isl[r], sis[r])
            pltpu.async_copy(dst_r.at[pl.ds(base, CHUNK)], idl[r], sis[r])

        def _wait_idx(r):
            pltpu.make_async_copy(src_r.at[pl.ds(e0, CHUNK)], isl[r], sis[r]).wait()
            pltpu.make_async_copy(dst_r.at[pl.ds(e0, CHUNK)], idl[r], sis[r]).wait()

        def _issue_gathers(b, r):
            rows, asr, adr, sg = dbufs[b]
            pltpu.async_copy(h_r.at[isl[r]], rows, sg)
            pltpu.async_copy(as_r.at[isl[r]], asr, sg)
            pltpu.async_copy(ad_r.at[idl[r]], adr, sg)

        def _wait_gathers(b, r):
            rows, asr, adr, sg = dbufs[b]
            pltpu.make_async_copy(h_r.at[isl[r]], rows, sg).wait()
            pltpu.make_async_copy(as_r.at[isl[r]], asr, sg).wait()
            pltpu.make_async_copy(ad_r.at[idl[r]], adr, sg).wait()

        def _scatter(b, r):
            rows, asr, _, _ = dbufs[b]
            pltpu.sync_copy(rows, accm_sh.at[idl[r]], add=True)
            pltpu.sync_copy(asr, accw_sh.at[idl[r]], add=True)

        def _compute(b):
            rows, asr, adr, _ = dbufs[b]

            def _edge(e, carry2):
                wrow = jnp.exp(_leaky(asr[e, :] + adr[e, :]))
                asr[e, :] = wrow
                for jb in range(F // 16):
                    s = wrow[jb * 16 // HID]
                    rows[e, pl.ds(jb * 16, 16)] = rows[e, pl.ds(jb * 16, 16)] * s
                return carry2
            lax.fori_loop(0, CHUNK, _edge, 0)

        _issue_idx(0, 0)
        _issue_idx(1, 1)
        _issue_idx(2, 2)
        _wait_idx(0)
        _issue_gathers(0, 0)

        def _quad(q, carry):
            for u in range(4):
                j = q * 4 + u
                b, o = u % 2, 1 - u % 2
                r, rn, rp = u % 4, (u + 1) % 4, (u + 3) % 4
                _wait_gathers(b, r)
                _issue_idx(jnp.minimum(j + 3, CPW - 1), rp)
                _wait_idx(rn)
                _issue_gathers(o, rn)
                _compute(b)
                _scatter(b, r)
            return carry
        lax.fori_loop(0, CPW // 4, _quad, 0)

        _wait_gathers(0, 0)
        _wait_idx(1)
        _wait_idx(2)

        plsc.subcore_barrier()
        r0 = sid * ROWS_PER_SUB
        pltpu.sync_copy(accm_sh.at[pl.ds(r0, ROWS_PER_SUB)],
                        accm_o.at[cid, pl.ds(r0, ROWS_PER_SUB)])
        pltpu.sync_copy(accw_sh.at[pl.ds(r0, ROWS_PER_SUB)],
                        accw_o.at[cid, pl.ds(r0, ROWS_PER_SUB)])

    return k(H, As, Ad, srcp, dstp)


def _mid_body(am_ref, aw_ref, h_ref, as_ref, ad_ref, erep_ref, b_ref,
              w2_ref, ms_ref, md_ref, h2_ref, as2_ref, ad2_ref):
    wself = jnp.exp(_leaky(as_ref[...] + ad_ref[...]))
    erep = erep_ref[...]
    den = jnp.dot(aw_ref[0] + aw_ref[1] + wself, erep,
                  preferred_element_type=jnp.float32)
    num = (am_ref[0] + am_ref[1]
           + jnp.dot(wself, erep, preferred_element_type=jnp.float32)
           * h_ref[...])
    x2 = _elu(num / den + b_ref[...])
    h2 = jnp.dot(x2, w2_ref[...], preferred_element_type=jnp.float32)
    h2_ref[...] = h2
    as2_ref[...] = jnp.dot(h2, ms_ref[...], preferred_element_type=jnp.float32)
    ad2_ref[...] = jnp.dot(h2, md_ref[...], preferred_element_type=jnp.float32)


def _mid(accm, accw, H1, As1, Ad1, Erep, b1, W2, Msrc2, Mdst2):
    blk = 1024
    grid = N_PAD // blk
    return pl.pallas_call(
        _mid_body,
        grid=(grid,),
        in_specs=[
            pl.BlockSpec((2, blk, F), lambda i: (0, i, 0)),
            pl.BlockSpec((2, blk, 16), lambda i: (0, i, 0)),
            pl.BlockSpec((blk, F), lambda i: (i, 0)),
            pl.BlockSpec((blk, 16), lambda i: (i, 0)),
            pl.BlockSpec((blk, 16), lambda i: (i, 0)),
            pl.BlockSpec((16, F), lambda i: (0, 0)),
            pl.BlockSpec((1, F), lambda i: (0, 0)),
            pl.BlockSpec((F, F), lambda i: (0, 0)),
            pl.BlockSpec((F, 16), lambda i: (0, 0)),
            pl.BlockSpec((F, 16), lambda i: (0, 0)),
        ],
        out_specs=[
            pl.BlockSpec((blk, F), lambda i: (i, 0)),
            pl.BlockSpec((blk, 16), lambda i: (i, 0)),
            pl.BlockSpec((blk, 16), lambda i: (i, 0)),
        ],
        out_shape=[
            jax.ShapeDtypeStruct((N_PAD, F), jnp.float32),
            jax.ShapeDtypeStruct((N_PAD, 16), jnp.float32),
            jax.ShapeDtypeStruct((N_PAD, 16), jnp.float32),
        ],
    )(accm, accw, H1, As1, Ad1, Erep, b1, W2, Msrc2, Mdst2)


def _final_body(am_ref, aw_ref, h_ref, as_ref, ad_ref, erep_ref, mavg_ref,
                b_ref, wc_ref, bc_ref, out_ref):
    wself = jnp.exp(_leaky(as_ref[...] + ad_ref[...]))
    erep = erep_ref[...]
    den = jnp.dot(aw_ref[0] + aw_ref[1] + wself, erep,
                  preferred_element_type=jnp.float32)
    num = (am_ref[0] + am_ref[1]
           + jnp.dot(wself, erep, preferred_element_type=jnp.float32)
           * h_ref[...])
    v = num / den
    y = _elu(jnp.dot(v, mavg_ref[...], preferred_element_type=jnp.float32)
             + b_ref[...])
    out_ref[...] = (jnp.dot(y, wc_ref[...], preferred_element_type=jnp.float32)
                    + bc_ref[...])


def _final(accm, accw, H2, As2, Ad2, Erep, Mavg, b2, Wc, bc):
    blk = 1000
    grid = N // blk
    return pl.pallas_call(
        _final_body,
        grid=(grid,),
        in_specs=[
            pl.BlockSpec((2, blk, F), lambda i: (0, i, 0)),
            pl.BlockSpec((2, blk, 16), lambda i: (0, i, 0)),
            pl.BlockSpec((blk, F), lambda i: (i, 0)),
            pl.BlockSpec((blk, 16), lambda i: (i, 0)),
            pl.BlockSpec((blk, 16), lambda i: (i, 0)),
            pl.BlockSpec((16, F), lambda i: (0, 0)),
            pl.BlockSpec((F, HID), lambda i: (0, 0)),
            pl.BlockSpec((1, HID), lambda i: (0, 0)),
            pl.BlockSpec((HID, 16), lambda i: (0, 0)),
            pl.BlockSpec((1, 16), lambda i: (0, 0)),
        ],
        out_specs=pl.BlockSpec((blk, 16), lambda i: (i, 0)),
        out_shape=jax.ShapeDtypeStruct((N, 16), jnp.float32),
    )(accm, accw, H2, As2, Ad2, Erep, Mavg, b2, Wc, bc)


def kernel(x, edge_index, W1, att_src1, att_dst1, b1,
           W2, att_src2, att_dst2, b2, Wc, bc):
    f32 = jnp.float32
    hm = (jnp.arange(16)[None, :] == (jnp.arange(F) // HID)[:, None]).astype(f32)
    Erep = hm.T
    Mavg = jnp.tile(jnp.eye(HID, dtype=f32), (HEADS, 1)) * (1.0 / HEADS)
    Msrc1 = att_src1.reshape(F)[:, None] * hm
    Mdst1 = att_dst1.reshape(F)[:, None] * hm
    Msrc2 = att_src2.reshape(F)[:, None] * hm
    Mdst2 = att_dst2.reshape(F)[:, None] * hm

    x_pad = jnp.pad(x, ((0, N_PAD - N), (0, 0)))
    pad_idx = jnp.full((E_PAD - E,), N, jnp.int32)
    srcp = jnp.concatenate([edge_index[0], pad_idx])
    dstp = jnp.concatenate([edge_index[1], pad_idx])
    H1, As1, Ad1 = _prologue(x_pad, W1, Msrc1, Mdst1)
    accm1, accw1 = _edge_pass(H1, As1, Ad1, srcp, dstp)
    H2, As2, Ad2 = _mid(accm1, accw1, H1, As1, Ad1, Erep,
                        b1.reshape(1, F), W2, Msrc2, Mdst2)
    accm2, accw2 = _edge_pass(H2, As2, Ad2, srcp, dstp)
    return _final(accm2, accw2, H2, As2, Ad2, Erep, Mavg,
                  b2.reshape(1, HID), Wc, bc.reshape(1, 16))
